# baseline (speedup 1.0000x reference)
import functools
import jax
import jax.numpy as jnp
from jax import lax
from jax.experimental import pallas as pl
from jax.experimental.pallas import tpu as pltpu
from jax.experimental.pallas import tpu_sc as plsc

N = 320000
D = 128
C = 128
G = 128
NGROUPS = N // G
NW = 32
GP_LO = NGROUPS // NW
GP_REM = NGROUPS % NW


def _sc_segsum(feats_hbm, lbl2d_hbm, acc_out, cnt_out,
               rows_v, lbl_v, acc_v, cnt_v, sem):
    cid = lax.axis_index("c")
    sid = lax.axis_index("s")
    wid = cid * 16 + sid

    zero16 = jnp.zeros((16,), jnp.float32)
    one16 = jnp.ones((16,), jnp.float32)

    def _zacc(r, _):
        for k in range(D // 16):
            acc_v[r, pl.ds(k * 16, 16)] = zero16
        cnt_v[r, :] = zero16
        return 0

    lax.fori_loop(0, C, _zacc, 0)

    start = wid * GP_LO + jnp.minimum(wid, GP_REM)
    count = GP_LO + jnp.where(wid < GP_REM, 1, 0)

    def _group(g, _):
        grp = start + g
        pltpu.sync_copy(lbl2d_hbm.at[grp], lbl_v)
        pltpu.sync_copy(feats_hbm.at[pl.ds(grp * G, G)], rows_v)

        def _row16(j, _):
            lab16 = lbl_v[pl.ds(j * 16, 16)]
            for t in range(16):
                lab = lab16[t]
                r = j * 16 + t
                for k in range(D // 16):
                    plsc.addupdate(acc_v.at[lab, pl.ds(k * 16, 16)],
                                   rows_v[r, pl.ds(k * 16, 16)])
                plsc.addupdate(cnt_v.at[lab], one16)
            return 0

        lax.fori_loop(0, G // 16, _row16, 0)
        return 0

    lax.fori_loop(0, count, _group, 0)

    pltpu.sync_copy(acc_v, acc_out.at[wid])
    pltpu.sync_copy(cnt_v, cnt_out.at[wid])


def _combine_kernel(acc_ref, cnt_ref, proto_ref, cov_ref, out_ref):
    sums = jnp.sum(acc_ref[...], axis=0)
    counts = jnp.sum(cnt_ref[...], axis=0)[:, 0:1]
    means = sums / jnp.maximum(counts, 1.0)
    present = (counts > 0.0).astype(jnp.float32)
    per_elem = (means - proto_ref[...]) ** 2 / (cov_ref[...] + 1e-6)
    per_elem = per_elem * present
    loss = jnp.sum(per_elem) / (jnp.sum(present) * D)
    out_ref[...] = jnp.reshape(loss, (1, 1))


def kernel(feats, pseudo_lbls, src_prototype, src_prototype_cov):
    lbl2d = jnp.reshape(pseudo_lbls, (NGROUPS, G))
    mesh = plsc.VectorSubcoreMesh(core_axis_name="c", subcore_axis_name="s")
    sc_call = functools.partial(
        pl.kernel, mesh=mesh,
        out_type=[
            jax.ShapeDtypeStruct((NW, C, D), jnp.float32),
            jax.ShapeDtypeStruct((NW, C, 16), jnp.float32),
        ],
        scratch_types=[
            pltpu.VMEM((G, D), jnp.float32),
            pltpu.VMEM((G,), jnp.int32),
            pltpu.VMEM((C, D), jnp.float32),
            pltpu.VMEM((C, 16), jnp.float32),
            pltpu.SemaphoreType.DMA,
        ],
    )(_sc_segsum)
    accw, cntw = sc_call(feats, lbl2d)

    out = pl.pallas_call(
        _combine_kernel,
        in_specs=[
            pl.BlockSpec((NW, C, D), lambda: (0, 0, 0)),
            pl.BlockSpec((NW, C, 16), lambda: (0, 0, 0)),
            pl.BlockSpec((C, D), lambda: (0, 0)),
            pl.BlockSpec((C, D), lambda: (0, 0)),
        ],
        out_specs=pl.BlockSpec((1, 1), lambda: (0, 0)),
        out_shape=jax.ShapeDtypeStruct((1, 1), jnp.float32),
    )(accw, cntw, src_prototype, src_prototype_cov)
    return out[0, 0]


# speedup vs baseline: 2.5923x; 2.5923x over previous
import functools
import jax
import jax.numpy as jnp
from jax import lax
from jax.experimental import pallas as pl
from jax.experimental.pallas import tpu as pltpu
from jax.experimental.pallas import tpu_sc as plsc

N = 320000
D = 128
C = 128

BLK = 32000
N_SC = 32000
N_TC = N - N_SC
GRID = N_TC // BLK

G = 128
NGROUPS = N // G
G0 = N_TC // G
NG_SC = N_SC // G
NW = 32
GP_LO = NG_SC // NW
GP_REM = NG_SC % NW


def _tc_partial(lbl_ref, feats_ref, acc_ref, cnt_ref):
    i = pl.program_id(0)
    lbl = lbl_ref[0]
    oh_t = (jnp.broadcast_to(lbl, (C, BLK)) ==
            jax.lax.broadcasted_iota(jnp.int32, (C, BLK), 0)
            ).astype(jnp.float32)
    feats = feats_ref[...]
    partial = jnp.dot(oh_t, feats, preferred_element_type=jnp.float32)
    ones = jnp.ones((BLK, 8), jnp.float32)
    pcnt = jnp.dot(oh_t, ones, preferred_element_type=jnp.float32)

    @pl.when(i == 0)
    def _init():
        acc_ref[...] = partial
        cnt_ref[...] = pcnt

    @pl.when(i > 0)
    def _accum():
        acc_ref[...] += partial
        cnt_ref[...] += pcnt


def _sc_segsum(feats_hbm, lbl2d_hbm, acc_out, cnt_out,
               rows_v, lbl_v, acc_v, cnt_v, sem):
    cid = lax.axis_index("c")
    sid = lax.axis_index("s")
    wid = cid * 16 + sid

    zero16 = jnp.zeros((16,), jnp.float32)
    one16 = jnp.ones((16,), jnp.float32)

    def _zacc(r, _):
        for k in range(D // 16):
            acc_v[r, pl.ds(k * 16, 16)] = zero16
        cnt_v[r, :] = zero16
        return 0

    lax.fori_loop(0, C, _zacc, 0)

    start = G0 + wid * GP_LO + jnp.minimum(wid, GP_REM)
    count = GP_LO + jnp.where(wid < GP_REM, 1, 0)

    def _group(g, _):
        grp = start + g
        pltpu.sync_copy(lbl2d_hbm.at[grp], lbl_v)
        pltpu.sync_copy(feats_hbm.at[pl.ds(grp * G, G)], rows_v)

        def _row16(j, _):
            lab16 = lbl_v[pl.ds(j * 16, 16)]
            for t in range(16):
                lab = lab16[t]
                r = j * 16 + t
                for k in range(D // 16):
                    plsc.addupdate(acc_v.at[lab, pl.ds(k * 16, 16)],
                                   rows_v[r, pl.ds(k * 16, 16)])
                plsc.addupdate(cnt_v.at[lab], one16)
            return 0

        lax.fori_loop(0, G // 16, _row16, 0)
        return 0

    lax.fori_loop(0, count, _group, 0)

    pltpu.sync_copy(acc_v, acc_out.at[wid])
    pltpu.sync_copy(cnt_v, cnt_out.at[wid])


def _combine_kernel(acc_tc_ref, cnt_tc_ref, accw_ref, cntw_ref, proto_ref,
                    cov_ref, out_ref):
    sums = acc_tc_ref[...] + jnp.sum(accw_ref[...], axis=0)
    counts = cnt_tc_ref[:, 0:1] + jnp.sum(cntw_ref[...], axis=0)[:, 0:1]
    means = sums / jnp.maximum(counts, 1.0)
    present = (counts > 0.0).astype(jnp.float32)
    per_elem = (means - proto_ref[...]) ** 2 / (cov_ref[...] + 1e-6)
    per_elem = per_elem * present
    loss = jnp.sum(per_elem) / (jnp.sum(present) * D)
    out_ref[...] = jnp.reshape(loss, (1, 1))


def kernel(feats, pseudo_lbls, src_prototype, src_prototype_cov):
    lbl2d = jnp.reshape(pseudo_lbls, (NGROUPS, G))
    lbls3 = jnp.reshape(pseudo_lbls[:N_TC], (GRID, 1, BLK))

    mesh = plsc.VectorSubcoreMesh(core_axis_name="c", subcore_axis_name="s")
    sc_call = functools.partial(
        pl.kernel, mesh=mesh,
        out_type=[
            jax.ShapeDtypeStruct((NW, C, D), jnp.float32),
            jax.ShapeDtypeStruct((NW, C, 16), jnp.float32),
        ],
        scratch_types=[
            pltpu.VMEM((G, D), jnp.float32),
            pltpu.VMEM((G,), jnp.int32),
            pltpu.VMEM((C, D), jnp.float32),
            pltpu.VMEM((C, 16), jnp.float32),
            pltpu.SemaphoreType.DMA,
        ],
    )(_sc_segsum)
    accw, cntw = sc_call(feats, lbl2d)

    acc_tc, cnt_tc = pl.pallas_call(
        _tc_partial,
        grid=(GRID,),
        in_specs=[
            pl.BlockSpec((1, 1, BLK), lambda i: (i, 0, 0)),
            pl.BlockSpec((BLK, D), lambda i: (i, 0)),
        ],
        out_specs=[
            pl.BlockSpec((C, D), lambda i: (0, 0)),
            pl.BlockSpec((C, 8), lambda i: (0, 0)),
        ],
        out_shape=[
            jax.ShapeDtypeStruct((C, D), jnp.float32),
            jax.ShapeDtypeStruct((C, 8), jnp.float32),
        ],
    )(lbls3, feats[:N_TC])

    out = pl.pallas_call(
        _combine_kernel,
        in_specs=[
            pl.BlockSpec((C, D), lambda: (0, 0)),
            pl.BlockSpec((C, 8), lambda: (0, 0)),
            pl.BlockSpec((NW, C, D), lambda: (0, 0, 0)),
            pl.BlockSpec((NW, C, 16), lambda: (0, 0, 0)),
            pl.BlockSpec((C, D), lambda: (0, 0)),
            pl.BlockSpec((C, D), lambda: (0, 0)),
        ],
        out_specs=pl.BlockSpec((1, 1), lambda: (0, 0)),
        out_shape=jax.ShapeDtypeStruct((1, 1), jnp.float32),
    )(acc_tc, cnt_tc, accw, cntw, src_prototype, src_prototype_cov)
    return out[0, 0]


# speedup vs baseline: 5.7385x; 2.2137x over previous
import functools
import jax
import jax.numpy as jnp
from jax import lax
from jax.experimental import pallas as pl
from jax.experimental.pallas import tpu as pltpu
from jax.experimental.pallas import tpu_sc as plsc

N = 320000
D = 128
C = 128

BLK = 32000
N_SC = 32000
N_TC = N - N_SC
GRID = N_TC // BLK

G = 128
NGROUPS = N // G
G0 = N_TC // G
NG_SC = N_SC // G
NW = 32
GP_LO = NG_SC // NW
GP_REM = NG_SC % NW


def _tc_partial(lbl_ref, feats_ref, acc_ref, cnt_ref):
    i = pl.program_id(0)
    lbl = lbl_ref[0]
    oh_t = (jnp.broadcast_to(lbl, (C, BLK)) ==
            jax.lax.broadcasted_iota(jnp.int32, (C, BLK), 0)
            ).astype(jnp.float32)
    feats = feats_ref[...]
    partial = jnp.dot(oh_t, feats, preferred_element_type=jnp.float32)
    ones = jnp.ones((BLK, 8), jnp.float32)
    pcnt = jnp.dot(oh_t, ones, preferred_element_type=jnp.float32)

    @pl.when(i == 0)
    def _init():
        acc_ref[...] = partial
        cnt_ref[...] = pcnt

    @pl.when(i > 0)
    def _accum():
        acc_ref[...] += partial
        cnt_ref[...] += pcnt


def _sc_segsum(feats_hbm, lbl2d_hbm, acc_out, cnt_out,
               rows_v, lbl_v, acc_v, cnt_v, sem):
    cid = lax.axis_index("c")
    sid = lax.axis_index("s")
    wid = cid * 16 + sid

    zero16 = jnp.zeros((16,), jnp.float32)
    one16 = jnp.ones((16,), jnp.float32)

    def _zacc(r, _):
        for k in range(D // 16):
            acc_v[r, pl.ds(k * 16, 16)] = zero16
        cnt_v[r, :] = zero16
        return 0

    lax.fori_loop(0, C, _zacc, 0)

    start = G0 + wid * GP_LO + jnp.minimum(wid, GP_REM)
    count = GP_LO + jnp.where(wid < GP_REM, 1, 0)

    def _group(g, _):
        grp = start + g
        pltpu.sync_copy(lbl2d_hbm.at[grp], lbl_v)
        pltpu.sync_copy(feats_hbm.at[pl.ds(grp * G, G)], rows_v)

        def _row16(j, _):
            lab16 = lbl_v[pl.ds(j * 16, 16)]
            for t in range(16):
                lab = lab16[t]
                r = j * 16 + t
                for k in range(D // 16):
                    plsc.addupdate(acc_v.at[lab, pl.ds(k * 16, 16)],
                                   rows_v[r, pl.ds(k * 16, 16)])
                plsc.addupdate(cnt_v.at[lab], one16)
            return 0

        lax.fori_loop(0, G // 16, _row16, 0)
        return 0

    lax.fori_loop(0, count, _group, 0)

    pltpu.sync_copy(acc_v, acc_out.at[wid])
    pltpu.sync_copy(cnt_v, cnt_out.at[wid])


def _combine_kernel(acc_tc_ref, cnt_tc_ref, accw_ref, cntw_ref, proto_ref,
                    cov_ref, out_ref):
    sums = acc_tc_ref[...] + jnp.sum(accw_ref[...], axis=0)
    counts = cnt_tc_ref[:, 0:1] + jnp.sum(cntw_ref[...], axis=0)[:, 0:1]
    means = sums / jnp.maximum(counts, 1.0)
    present = (counts > 0.0).astype(jnp.float32)
    per_elem = (means - proto_ref[...]) ** 2 / (cov_ref[...] + 1e-6)
    per_elem = per_elem * present
    loss = jnp.sum(per_elem) / (jnp.sum(present) * D)
    out_ref[...] = jnp.reshape(loss, (1, 1))


def kernel(feats, pseudo_lbls, src_prototype, src_prototype_cov):
    lbl2d = jnp.reshape(pseudo_lbls, (NGROUPS, G))
    lbls3 = jnp.reshape(pseudo_lbls, (N // BLK, 1, BLK))

    mesh = plsc.VectorSubcoreMesh(core_axis_name="c", subcore_axis_name="s")
    sc_call = functools.partial(
        pl.kernel, mesh=mesh,
        out_type=[
            jax.ShapeDtypeStruct((NW, C, D), jnp.float32),
            jax.ShapeDtypeStruct((NW, C, 16), jnp.float32),
        ],
        scratch_types=[
            pltpu.VMEM((G, D), jnp.float32),
            pltpu.VMEM((G,), jnp.int32),
            pltpu.VMEM((C, D), jnp.float32),
            pltpu.VMEM((C, 16), jnp.float32),
            pltpu.SemaphoreType.DMA,
        ],
    )(_sc_segsum)
    accw, cntw = sc_call(feats, lbl2d)

    acc_tc, cnt_tc = pl.pallas_call(
        _tc_partial,
        grid=(GRID,),
        in_specs=[
            pl.BlockSpec((1, 1, BLK), lambda i: (i, 0, 0)),
            pl.BlockSpec((BLK, D), lambda i: (i, 0)),
        ],
        out_specs=[
            pl.BlockSpec((C, D), lambda i: (0, 0)),
            pl.BlockSpec((C, 8), lambda i: (0, 0)),
        ],
        out_shape=[
            jax.ShapeDtypeStruct((C, D), jnp.float32),
            jax.ShapeDtypeStruct((C, 8), jnp.float32),
        ],
    )(lbls3, feats)

    out = pl.pallas_call(
        _combine_kernel,
        in_specs=[
            pl.BlockSpec((C, D), lambda: (0, 0)),
            pl.BlockSpec((C, 8), lambda: (0, 0)),
            pl.BlockSpec((NW, C, D), lambda: (0, 0, 0)),
            pl.BlockSpec((NW, C, 16), lambda: (0, 0, 0)),
            pl.BlockSpec((C, D), lambda: (0, 0)),
            pl.BlockSpec((C, D), lambda: (0, 0)),
        ],
        out_specs=pl.BlockSpec((1, 1), lambda: (0, 0)),
        out_shape=jax.ShapeDtypeStruct((1, 1), jnp.float32),
    )(acc_tc, cnt_tc, accw, cntw, src_prototype, src_prototype_cov)
    return out[0, 0]


# speedup vs baseline: 7.3445x; 1.2798x over previous
import jax
import jax.numpy as jnp
from jax.experimental import pallas as pl
from jax.experimental.pallas import tpu as pltpu

N = 320000
D = 128
C = 128
BLK = 32000
GRID = N // BLK


def _seg_loss_kernel(lbl_ref, feats_ref, proto_ref, cov_ref, out_ref,
                     acc_ref, cnt_ref):
    i = pl.program_id(0)
    lbl = lbl_ref[0]
    oh_t = (jnp.broadcast_to(lbl, (C, BLK)) ==
            jax.lax.broadcasted_iota(jnp.int32, (C, BLK), 0)
            ).astype(jnp.float32)
    feats = feats_ref[...]
    partial = jnp.dot(oh_t, feats, preferred_element_type=jnp.float32)
    ones = jnp.ones((BLK, 8), jnp.float32)
    pcnt = jnp.dot(oh_t, ones, preferred_element_type=jnp.float32)

    @pl.when(i == 0)
    def _init():
        acc_ref[...] = partial
        cnt_ref[...] = pcnt

    @pl.when(i > 0)
    def _accum():
        acc_ref[...] += partial
        cnt_ref[...] += pcnt

    @pl.when(i == GRID - 1)
    def _epilogue():
        counts = cnt_ref[:, 0:1]
        means = acc_ref[...] / jnp.maximum(counts, 1.0)
        present = (counts > 0.0).astype(jnp.float32)
        per_elem = (means - proto_ref[...]) ** 2 / (cov_ref[...] + 1e-6)
        per_elem = per_elem * present
        loss = jnp.sum(per_elem) / (jnp.sum(present) * D)
        out_ref[...] = jnp.reshape(loss, (1, 1))


def kernel(feats, pseudo_lbls, src_prototype, src_prototype_cov):
    lbls3 = jnp.reshape(pseudo_lbls, (GRID, 1, BLK))
    out = pl.pallas_call(
        _seg_loss_kernel,
        grid=(GRID,),
        in_specs=[
            pl.BlockSpec((1, 1, BLK), lambda i: (i, 0, 0)),
            pl.BlockSpec((BLK, D), lambda i: (i, 0)),
            pl.BlockSpec((C, D), lambda i: (0, 0)),
            pl.BlockSpec((C, D), lambda i: (0, 0)),
        ],
        out_specs=pl.BlockSpec((1, 1), lambda i: (0, 0)),
        out_shape=jax.ShapeDtypeStruct((1, 1), jnp.float32),
        scratch_shapes=[
            pltpu.VMEM((C, D), jnp.float32),
            pltpu.VMEM((C, 8), jnp.float32),
        ],
    )(lbls3, feats, src_prototype, src_prototype_cov)
    return out[0, 0]
